# baseline (device time: 64494 ns/iter reference)
import jax
import jax.numpy as jnp
from jax import lax
from jax.experimental import pallas as pl
from jax.experimental.pallas import tpu as pltpu

N_DEV = 16
N_STEPS = 4
DIMS = ("x", "y", "z1", "z2")

GROUPS = (
    (0, 1024, ("x", "y", "z1", "z2")),
)


def kernel(x):
    m, n = x.shape
    n_g = len(GROUPS)

    def body(x_ref, out_ref, *scratch):
        comm = scratch[: n_g * N_STEPS]
        send_sems, recv_sems = scratch[n_g * N_STEPS :]

        my = lax.axis_index("i")
        p = my % 4
        z = my // 4
        coord = {
            "x": (p ^ (p >> 1)) & 1,
            "y": p >> 1,
            "z1": z & 1,
            "z2": (z >> 1) & 1,
        }
        partner = {
            "x": 4 * z + (p ^ 1),
            "y": 4 * z + (p ^ 3),
            "z1": 4 * (z ^ 1) + p,
            "z2": 4 * (z ^ 2) + p,
        }

        barrier_sem = pltpu.get_barrier_semaphore()
        for d in DIMS:
            pl.semaphore_signal(
                barrier_sem, inc=1,
                device_id=(partner[d],), device_id_type=pl.DeviceIdType.MESH,
            )
        pl.semaphore_wait(barrier_sem, 4)

        offs = [g0 for (g0, _, _) in GROUPS]
        for k in range(N_STEPS):
            rdmas, keeps = [], []
            for gi, (g0, r, order) in enumerate(GROUPS):
                d = order[k]
                h = r >> (k + 1)
                bit = coord[d]
                keep_off = offs[gi] + bit * h
                send_off = offs[gi] + (1 - bit) * h
                src = x_ref if k == 0 else out_ref
                rdma = pltpu.make_async_remote_copy(
                    src_ref=src.at[pl.ds(send_off, h)],
                    dst_ref=comm[gi * N_STEPS + k],
                    send_sem=send_sems.at[gi, k],
                    recv_sem=recv_sems.at[gi, k],
                    device_id=(partner[d],),
                    device_id_type=pl.DeviceIdType.MESH,
                )
                rdma.start()
                rdmas.append(rdma)
                keeps.append((keep_off, h))
            for gi in range(n_g):
                rdmas[gi].wait()
            for gi, (g0, r, order) in enumerate(GROUPS):
                keep_off, h = keeps[gi]
                src = x_ref if k == 0 else out_ref
                out_ref[pl.ds(keep_off, h), :] = (
                    src[pl.ds(keep_off, h), :] + comm[gi * N_STEPS + k][:, :]
                )
                offs[gi] = keep_off

        sizes = [r >> N_STEPS for (_, r, _) in GROUPS]
        for k in range(N_STEPS):
            rdmas = []
            for gi, (g0, r, order) in enumerate(GROUPS):
                d = order[N_STEPS - 1 - k]
                s = sizes[gi]
                rdma = pltpu.make_async_remote_copy(
                    src_ref=out_ref.at[pl.ds(offs[gi], s)],
                    dst_ref=out_ref.at[pl.ds(offs[gi], s)],
                    send_sem=send_sems.at[gi, N_STEPS + k],
                    recv_sem=recv_sems.at[gi, N_STEPS + k],
                    device_id=(partner[d],),
                    device_id_type=pl.DeviceIdType.MESH,
                )
                rdma.start()
                rdmas.append(rdma)
            for gi, (g0, r, order) in enumerate(GROUPS):
                rdmas[gi].wait()
                d = order[N_STEPS - 1 - k]
                offs[gi] = offs[gi] - coord[d] * sizes[gi]
                sizes[gi] = sizes[gi] * 2

    comm_shapes = [
        pltpu.VMEM((r >> (k + 1), n), x.dtype)
        for (_, r, _) in GROUPS
        for k in range(N_STEPS)
    ]
    return pl.pallas_call(
        body,
        out_shape=jax.ShapeDtypeStruct((m, n), x.dtype),
        in_specs=[pl.BlockSpec(memory_space=pltpu.VMEM)],
        out_specs=pl.BlockSpec(memory_space=pltpu.VMEM),
        scratch_shapes=comm_shapes + [
            pltpu.SemaphoreType.DMA((n_g, 2 * N_STEPS)),
            pltpu.SemaphoreType.DMA((n_g, 2 * N_STEPS)),
        ],
        compiler_params=pltpu.CompilerParams(collective_id=0),
    )(x)


# device time: 39809 ns/iter; 1.6201x vs baseline; 1.6201x over previous
import jax
import jax.numpy as jnp
from jax import lax
from jax.experimental import pallas as pl
from jax.experimental.pallas import tpu as pltpu

N_DEV = 16
N_STEPS = 4
DIMS = ("x", "y", "z1", "z2")

GROUPS = (
    (0, 384, ("x", "y", "z1", "z2")),
    (384, 384, ("y", "x", "z2", "z1")),
    (768, 128, ("z1", "z2", "x", "y")),
    (896, 128, ("z2", "z1", "y", "x")),
)


def kernel(x):
    m, n = x.shape
    n_g = len(GROUPS)

    def body(x_ref, out_ref, *scratch):
        comm = scratch[: n_g * N_STEPS]
        send_sems, recv_sems = scratch[n_g * N_STEPS :]

        my = lax.axis_index("i")
        p = my % 4
        z = my // 4
        coord = {
            "x": (p ^ (p >> 1)) & 1,
            "y": p >> 1,
            "z1": z & 1,
            "z2": (z >> 1) & 1,
        }
        partner = {
            "x": 4 * z + (p ^ 1),
            "y": 4 * z + (p ^ 3),
            "z1": 4 * (z ^ 1) + p,
            "z2": 4 * (z ^ 2) + p,
        }

        def rs_copy(gi, k, src_ref, src_off, h, d):
            rdma = pltpu.make_async_remote_copy(
                src_ref=src_ref.at[pl.ds(src_off, h)],
                dst_ref=comm[gi * N_STEPS + k],
                send_sem=send_sems.at[gi, k],
                recv_sem=recv_sems.at[gi, k],
                device_id=(partner[d],),
                device_id_type=pl.DeviceIdType.MESH,
            )
            rdma.start()
            return rdma

        def ag_copy(gi, k, off, s, d):
            rdma = pltpu.make_async_remote_copy(
                src_ref=out_ref.at[pl.ds(off, s)],
                dst_ref=out_ref.at[pl.ds(off, s)],
                send_sem=send_sems.at[gi, N_STEPS + k],
                recv_sem=recv_sems.at[gi, N_STEPS + k],
                device_id=(partner[d],),
                device_id_type=pl.DeviceIdType.MESH,
            )
            rdma.start()
            return rdma

        barrier_sem = pltpu.get_barrier_semaphore()
        for d in DIMS:
            pl.semaphore_signal(
                barrier_sem, inc=1,
                device_id=(partner[d],), device_id_type=pl.DeviceIdType.MESH,
            )
        pl.semaphore_wait(barrier_sem, 4)

        offs, rs_rdmas, ag_rdmas = [], [], [None] * n_g
        for gi, (g0, r, order) in enumerate(GROUPS):
            d = order[0]
            h = r >> 1
            send_off = g0 + (1 - coord[d]) * h
            rs_rdmas.append(rs_copy(gi, 0, x_ref, send_off, h, d))
            offs.append(g0)

        for k in range(N_STEPS):
            for gi, (g0, r, order) in enumerate(GROUPS):
                h = r >> (k + 1)
                keep_off = offs[gi] + coord[order[k]] * h
                rs_rdmas[gi].wait()
                src = x_ref if k == 0 else out_ref
                cm = comm[gi * N_STEPS + k]
                if k < N_STEPS - 1:
                    d2 = order[k + 1]
                    h2 = h >> 1
                    bit2 = coord[d2]
                    sn_rel = (1 - bit2) * h2
                    kn_rel = bit2 * h2
                    sn = keep_off + sn_rel
                    kn = keep_off + kn_rel
                    out_ref[pl.ds(sn, h2), :] = (
                        src[pl.ds(sn, h2), :] + cm[pl.ds(sn_rel, h2), :]
                    )
                    rs_rdmas[gi] = rs_copy(gi, k + 1, out_ref, sn, h2, d2)
                    out_ref[pl.ds(kn, h2), :] = (
                        src[pl.ds(kn, h2), :] + cm[pl.ds(kn_rel, h2), :]
                    )
                else:
                    out_ref[pl.ds(keep_off, h), :] = (
                        src[pl.ds(keep_off, h), :] + cm[:, :]
                    )
                    ag_rdmas[gi] = ag_copy(
                        gi, 0, keep_off, r >> N_STEPS, order[N_STEPS - 1]
                    )
                offs[gi] = keep_off

        sizes = [r >> N_STEPS for (_, r, _) in GROUPS]
        for k in range(N_STEPS):
            for gi, (g0, r, order) in enumerate(GROUPS):
                ag_rdmas[gi].wait()
                d = order[N_STEPS - 1 - k]
                offs[gi] = offs[gi] - coord[d] * sizes[gi]
                sizes[gi] = sizes[gi] * 2
                if k < N_STEPS - 1:
                    ag_rdmas[gi] = ag_copy(
                        gi, k + 1, offs[gi], sizes[gi],
                        order[N_STEPS - 2 - k],
                    )

    comm_shapes = [
        pltpu.VMEM((r >> (k + 1), n), x.dtype)
        for (_, r, _) in GROUPS
        for k in range(N_STEPS)
    ]
    return pl.pallas_call(
        body,
        out_shape=jax.ShapeDtypeStruct((m, n), x.dtype),
        in_specs=[pl.BlockSpec(memory_space=pltpu.VMEM)],
        out_specs=pl.BlockSpec(memory_space=pltpu.VMEM),
        scratch_shapes=comm_shapes + [
            pltpu.SemaphoreType.DMA((n_g, 2 * N_STEPS)),
            pltpu.SemaphoreType.DMA((n_g, 2 * N_STEPS)),
        ],
        compiler_params=pltpu.CompilerParams(collective_id=0),
    )(x)


# device time: 38550 ns/iter; 1.6730x vs baseline; 1.0327x over previous
import jax
import jax.numpy as jnp
from jax import lax
from jax.experimental import pallas as pl
from jax.experimental.pallas import tpu as pltpu

N_DEV = 16
N_STEPS = 4
DIMS = ("x", "y", "z1", "z2")

GROUPS = (
    (0, 384, ("x", "y", "z1", "z2")),
    (384, 384, ("y", "x", "z2", "z1")),
    (768, 128, ("z1", "z2", "x", "y")),
    (896, 128, ("z2", "z1", "y", "x")),
)

G_ORDER = (2, 3, 0, 1)


def kernel(x):
    m, n = x.shape
    n_g = len(GROUPS)

    def body(x_ref, out_ref, *scratch):
        comm = scratch[: n_g * N_STEPS]
        send_sems, recv_sems = scratch[n_g * N_STEPS :]

        my = lax.axis_index("i")
        p = my % 4
        z = my // 4
        coord = {
            "x": (p ^ (p >> 1)) & 1,
            "y": p >> 1,
            "z1": z & 1,
            "z2": (z >> 1) & 1,
        }
        partner = {
            "x": 4 * z + (p ^ 1),
            "y": 4 * z + (p ^ 3),
            "z1": 4 * (z ^ 1) + p,
            "z2": 4 * (z ^ 2) + p,
        }

        def rs_copy(gi, k, half, src_ref, src_off, h, comm_rel, d):
            rdma = pltpu.make_async_remote_copy(
                src_ref=src_ref.at[pl.ds(src_off, h)],
                dst_ref=comm[gi * N_STEPS + k].at[pl.ds(comm_rel, h)],
                send_sem=send_sems.at[gi, 2 * k + half],
                recv_sem=recv_sems.at[gi, 2 * k + half],
                device_id=(partner[d],),
                device_id_type=pl.DeviceIdType.MESH,
            )
            rdma.start()
            return rdma

        def rs_send_step(gi, k, src_ref, keep_off, h, order):
            bit_k = coord[order[k]]
            send_off = keep_off + (1 - 2 * bit_k) * h
            if k == N_STEPS - 1:
                r = rs_copy(gi, k, 0, src_ref, send_off, h, 0, order[k])
                return [(r, 0, h)]
            h2 = h >> 1
            bit2 = coord[order[k + 1]]
            first_rel = (1 - bit2) * h2
            second_rel = bit2 * h2
            ra = rs_copy(gi, k, 0, src_ref, send_off + first_rel, h2,
                         first_rel, order[k])
            rb = rs_copy(gi, k, 1, src_ref, send_off + second_rel, h2,
                         second_rel, order[k])
            return [(ra, first_rel, h2), (rb, second_rel, h2)]

        def ag_copy(gi, k, off, s, d):
            rdma = pltpu.make_async_remote_copy(
                src_ref=out_ref.at[pl.ds(off, s)],
                dst_ref=out_ref.at[pl.ds(off, s)],
                send_sem=send_sems.at[gi, 2 * N_STEPS + k],
                recv_sem=recv_sems.at[gi, 2 * N_STEPS + k],
                device_id=(partner[d],),
                device_id_type=pl.DeviceIdType.MESH,
            )
            rdma.start()
            return rdma

        barrier_sem = pltpu.get_barrier_semaphore()
        for d in DIMS:
            pl.semaphore_signal(
                barrier_sem, inc=1,
                device_id=(partner[d],), device_id_type=pl.DeviceIdType.MESH,
            )
        pl.semaphore_wait(barrier_sem, 4)

        offs, rs_pend, ag_rdmas = [None] * n_g, [None] * n_g, [None] * n_g
        for gi in (0, 1, 2, 3):
            g0, r, order = GROUPS[gi]
            h = r >> 1
            keep_off = g0 + coord[order[0]] * h
            rs_pend[gi] = rs_send_step(gi, 0, x_ref, keep_off, h, order)
            offs[gi] = keep_off

        for k in range(N_STEPS):
            for gi in G_ORDER:
                g0, r, order = GROUPS[gi]
                h = r >> (k + 1)
                keep_off = offs[gi]
                src = x_ref if k == 0 else out_ref
                cm = comm[gi * N_STEPS + k]
                parts = rs_pend[gi]
                rdma, rel, hh = parts[0]
                rdma.wait()
                out_ref[pl.ds(keep_off + rel, hh), :] = (
                    src[pl.ds(keep_off + rel, hh), :]
                    + cm[pl.ds(rel, hh), :]
                )
                if k < N_STEPS - 1:
                    h2 = h >> 1
                    keep2 = keep_off + coord[order[k + 1]] * h2
                    rs_pend[gi] = rs_send_step(
                        gi, k + 1, out_ref, keep2, h2, order
                    )
                    offs[gi] = keep2
                else:
                    ag_rdmas[gi] = ag_copy(
                        gi, 0, keep_off, r >> N_STEPS, order[N_STEPS - 1]
                    )
                for rdma, rel, hh in parts[1:]:
                    rdma.wait()
                    out_ref[pl.ds(keep_off + rel, hh), :] = (
                        src[pl.ds(keep_off + rel, hh), :]
                        + cm[pl.ds(rel, hh), :]
                    )

        sizes = [r >> N_STEPS for (_, r, _) in GROUPS]
        for k in range(N_STEPS):
            for gi in G_ORDER:
                g0, r, order = GROUPS[gi]
                ag_rdmas[gi].wait()
                d = order[N_STEPS - 1 - k]
                offs[gi] = offs[gi] - coord[d] * sizes[gi]
                sizes[gi] = sizes[gi] * 2
                if k < N_STEPS - 1:
                    ag_rdmas[gi] = ag_copy(
                        gi, k + 1, offs[gi], sizes[gi],
                        order[N_STEPS - 2 - k],
                    )

    comm_shapes = [
        pltpu.VMEM((r >> (k + 1), n), x.dtype)
        for (_, r, _) in GROUPS
        for k in range(N_STEPS)
    ]
    return pl.pallas_call(
        body,
        out_shape=jax.ShapeDtypeStruct((m, n), x.dtype),
        in_specs=[pl.BlockSpec(memory_space=pltpu.VMEM)],
        out_specs=pl.BlockSpec(memory_space=pltpu.VMEM),
        scratch_shapes=comm_shapes + [
            pltpu.SemaphoreType.DMA((n_g, 2 * N_STEPS + N_STEPS)),
            pltpu.SemaphoreType.DMA((n_g, 2 * N_STEPS + N_STEPS)),
        ],
        compiler_params=pltpu.CompilerParams(collective_id=0),
    )(x)
